# baseline (speedup 1.0000x reference)
import functools

import jax
import jax.numpy as jnp
from jax import lax
from jax.experimental import pallas as pl
from jax.experimental.pallas import tpu as pltpu
from jax.experimental.pallas import tpu_sc as plsc

D = 128
NCH = 3
B = 4096
L = 200
SEGS = NCH * B
NC = 2
NS = 16
NW = NC * NS
SEG_PER_W = SEGS // NW
CHUNKS = 2
K = 100
LANES = D // 16
G = 8
NGRP = SEG_PER_W // G

_mesh = plsc.VectorSubcoreMesh(core_axis_name="c", subcore_axis_name="s")

VOCAB = 100000
BR = 125
NBLK = VOCAB // NW // BR
PU = 5


@functools.partial(
    pl.kernel,
    mesh=_mesh,
    out_type=jax.ShapeDtypeStruct((VOCAB, D // 2), jnp.int32),
    compiler_params=pltpu.CompilerParams(use_tc_tiling_on_sc=False),
    scratch_types=[
        pltpu.VMEM((BR, D), jnp.float32),
        pltpu.VMEM((BR, D), jnp.float32),
        pltpu.VMEM((BR, D // 2), jnp.int32),
        pltpu.VMEM((BR, D // 2), jnp.int32),
        pltpu.SemaphoreType.DMA,
        pltpu.SemaphoreType.DMA,
        pltpu.SemaphoreType.DMA,
        pltpu.SemaphoreType.DMA,
    ],
)
def _sc_pack(emb_hbm, out_hbm, in_a, in_b, pk_a, pk_b, sem_a, sem_b,
             osem_a, osem_b):
    wid = lax.axis_index("s") * NC + lax.axis_index("c")
    base = wid * (VOCAB // NW)
    ins = (in_a, in_b)
    pks = (pk_a, pk_b)
    sems = (sem_a, sem_b)
    osems = (osem_a, osem_b)
    cps = {0: pltpu.async_copy(emb_hbm.at[pl.ds(base, BR)], ins[0], sems[0])}
    ocps = {}
    for b in range(NBLK):
        if b + 1 < NBLK:
            cps[(b + 1) % 2] = pltpu.async_copy(
                emb_hbm.at[pl.ds(base + (b + 1) * BR, BR)], ins[(b + 1) % 2],
                sems[(b + 1) % 2])
        cps[b % 2].wait()
        if b >= 2:
            ocps[b % 2].wait()
        src = ins[b % 2]
        pk = pks[b % 2]

        def prow(r, carry):
            for rr in range(PU):
                for u in range(4):
                    row = PU * r + rr
                    wa = lax.bitcast_convert_type(
                        src[row, pl.ds(16 * u, 16)], jnp.int32)
                    wb = lax.bitcast_convert_type(
                        src[row, pl.ds(64 + 16 * u, 16)], jnp.int32)
                    ta = lax.shift_right_logical(wa + 0x8000, 16)
                    tb = (wb + 0x8000) & jnp.int32(-65536)
                    pk[row, pl.ds(16 * u, 16)] = ta | tb
            return carry

        lax.fori_loop(0, BR // PU, prow, 0)
        ocps[b % 2] = pltpu.async_copy(
            pk, out_hbm.at[pl.ds(base + b * BR, BR)], osems[b % 2])
    ocps[(NBLK - 2) % 2].wait()
    ocps[(NBLK - 1) % 2].wait()


@functools.partial(
    pl.kernel,
    mesh=_mesh,
    out_type=jax.ShapeDtypeStruct((SEGS, D), jnp.float32),
    compiler_params=pltpu.CompilerParams(use_tc_tiling_on_sc=False),
    scratch_types=[
        pltpu.VMEM((G, CHUNKS, K), jnp.int32),
        pltpu.VMEM((G, CHUNKS, K), jnp.int32),
        pltpu.VMEM((K, D // 2), jnp.int32),
        pltpu.VMEM((K, D // 2), jnp.int32),
        pltpu.VMEM((K, D // 2), jnp.int32),
        pltpu.VMEM((G, D), jnp.float32),
        pltpu.VMEM((G, D), jnp.float32),
        pltpu.SemaphoreType.DMA,
        pltpu.SemaphoreType.DMA,
        pltpu.SemaphoreType.DMA,
        pltpu.SemaphoreType.DMA,
        pltpu.SemaphoreType.DMA,
        pltpu.SemaphoreType.DMA,
        pltpu.SemaphoreType.DMA,
    ],
)
def _sc_pool(idx_hbm, emb_hbm, out_hbm, idx_a, idx_b, rows_a, rows_b, rows_c,
             ost_a, ost_b, sia, sib, sga, sgb, sgc, soa, sob):
    wid = lax.axis_index("s") * NC + lax.axis_index("c")
    base = wid * SEG_PER_W
    idxs = (idx_a, idx_b)
    rows = (rows_a, rows_b, rows_c)
    osts = (ost_a, ost_b)
    isems = (sia, sib)
    gsems = (sga, sgb, sgc)
    osems = (soa, sob)
    NCK = G * CHUNKS
    RU = 4
    icps = {}
    ocps = {}

    def process_group(g, q, first):
        s0 = base + g * G
        icps[q].wait()
        gn = jnp.minimum(g + 1, NGRP - 1)
        icps[1 - q] = pltpu.async_copy(
            idx_hbm.at[pl.ds(base + gn * G, G)], idxs[1 - q], isems[1 - q])
        if not first:
            ocps[q].wait()
        iv = idxs[q]
        ost = osts[q]
        cps = {
            0: pltpu.async_copy(emb_hbm.at[iv.at[0, 0]], rows[0], gsems[0]),
            1: pltpu.async_copy(emb_hbm.at[iv.at[0, 1]], rows[1], gsems[1]),
        }
        acc = None
        for t in range(NCK):
            seg, j = divmod(t, CHUNKS)
            if t + 2 < NCK:
                seg2, j2 = divmod(t + 2, CHUNKS)
                cps[(t + 2) % 3] = pltpu.async_copy(
                    emb_hbm.at[iv.at[seg2, j2]], rows[(t + 2) % 3],
                    gsems[(t + 2) % 3])
            cps[t % 3].wait()
            buf = rows[t % 3]
            if j == 0:
                acc = tuple(jnp.zeros((16,), jnp.float32) for _ in range(LANES))

            def red(m, a):
                a = list(a)
                for mm in range(RU):
                    for u in range(4):
                        w = buf[RU * m + mm, pl.ds(16 * u, 16)]
                        a[u] = a[u] + lax.bitcast_convert_type(
                            w << 16, jnp.float32)
                        a[4 + u] = a[4 + u] + lax.bitcast_convert_type(
                            w, jnp.float32)
                return tuple(a)

            acc = lax.fori_loop(0, K // RU, red, acc)
            if j == CHUNKS - 1:
                for u in range(4):
                    ost[seg, pl.ds(16 * u, 16)] = acc[u] * (1.0 / L)
                    ost[seg, pl.ds(64 + 16 * u, 16)] = acc[4 + u] * (1.0 / L)
        ocps[q] = pltpu.async_copy(ost, out_hbm.at[pl.ds(s0, G)], osems[q])

    icps[0] = pltpu.async_copy(idx_hbm.at[pl.ds(base, G)], idxs[0], isems[0])
    process_group(0, 0, True)
    process_group(1, 1, True)

    def body(p, carry):
        process_group(2 * p, 0, False)
        process_group(2 * p + 1, 1, False)
        return carry

    lax.fori_loop(1, NGRP // 2, body, 0)
    pltpu.make_async_copy(
        idx_hbm.at[pl.ds(base, G)], idxs[0], isems[0]).wait()
    for q in (0, 1):
        pltpu.make_async_copy(
            osts[q], out_hbm.at[pl.ds(base, G)], osems[q]).wait()


BB = 512
H = 256
OPAD = 128


def _mlp_body(p_ref, w1_ref, b1_ref, w2_ref, b2_ref, o_ref):
    p = p_ref[...]
    w1 = w1_ref[...]
    h = jnp.dot(p[0], w1[0:D], preferred_element_type=jnp.float32)
    h = h + jnp.dot(p[1], w1[D:2 * D], preferred_element_type=jnp.float32)
    h = h + jnp.dot(p[2], w1[2 * D:3 * D], preferred_element_type=jnp.float32)
    h = jnp.maximum(h + b1_ref[...], 0.0)
    o_ref[...] = jnp.dot(h, w2_ref[...],
                         preferred_element_type=jnp.float32) + b2_ref[...]


_mlp = pl.pallas_call(
    _mlp_body,
    grid=(B // BB,),
    in_specs=[
        pl.BlockSpec((NCH, BB, D), lambda i: (0, i, 0)),
        pl.BlockSpec((NCH * D, H), lambda i: (0, 0)),
        pl.BlockSpec((1, H), lambda i: (0, 0)),
        pl.BlockSpec((H, OPAD), lambda i: (0, 0)),
        pl.BlockSpec((1, OPAD), lambda i: (0, 0)),
    ],
    out_specs=pl.BlockSpec((BB, OPAD), lambda i: (i, 0)),
    out_shape=jax.ShapeDtypeStruct((B, OPAD), jnp.float32),
)


def kernel(x, emb, fc1_w, fc1_b, fc2_w, fc2_b):
    x = x.astype(jnp.int32)
    idx = jnp.concatenate([x[0], x[2], x[3]], axis=0).reshape(SEGS, CHUNKS, K)
    pooled = _sc_pool(idx, _sc_pack(emb))
    pooled3 = pooled.reshape(NCH, B, D)
    w1t = fc1_w.T
    b1 = fc1_b.reshape(1, H)
    w2t = jnp.zeros((H, OPAD), jnp.float32).at[:, :10].set(fc2_w.T)
    b2 = jnp.zeros((1, OPAD), jnp.float32).at[0, :10].set(fc2_b)
    out = _mlp(pooled3, w1t, b1, w2t, b2)
    return out[:, :10]


# speedup vs baseline: 1.0016x; 1.0016x over previous
import functools

import jax
import jax.numpy as jnp
from jax import lax
from jax.experimental import pallas as pl
from jax.experimental.pallas import tpu as pltpu
from jax.experimental.pallas import tpu_sc as plsc

D = 128
NCH = 3
B = 4096
L = 200
SEGS = NCH * B
NC = 2
NS = 16
NW = NC * NS
SEG_PER_W = SEGS // NW
CHUNKS = 2
K = 100
LANES = D // 16
G = 8
NGRP = SEG_PER_W // G

_mesh = plsc.VectorSubcoreMesh(core_axis_name="c", subcore_axis_name="s")

VOCAB = 100000
BR = 125
NBLK = VOCAB // NW // BR
PU = 5


@functools.partial(
    pl.kernel,
    mesh=_mesh,
    out_type=jax.ShapeDtypeStruct((VOCAB, D // 2), jnp.int32),
    compiler_params=pltpu.CompilerParams(use_tc_tiling_on_sc=False),
    scratch_types=[
        pltpu.VMEM((BR, D), jnp.float32),
        pltpu.VMEM((BR, D), jnp.float32),
        pltpu.VMEM((BR, D), jnp.float32),
        pltpu.VMEM((BR, D // 2), jnp.int32),
        pltpu.VMEM((BR, D // 2), jnp.int32),
        pltpu.SemaphoreType.DMA,
        pltpu.SemaphoreType.DMA,
        pltpu.SemaphoreType.DMA,
        pltpu.SemaphoreType.DMA,
        pltpu.SemaphoreType.DMA,
    ],
)
def _sc_pack(emb_hbm, out_hbm, in_a, in_b, in_c, pk_a, pk_b, sem_a, sem_b,
             sem_c, osem_a, osem_b):
    wid = lax.axis_index("s") * NC + lax.axis_index("c")
    base = wid * (VOCAB // NW)
    ins = (in_a, in_b, in_c)
    pks = (pk_a, pk_b)
    sems = (sem_a, sem_b, sem_c)
    osems = (osem_a, osem_b)
    cps = {
        0: pltpu.async_copy(emb_hbm.at[pl.ds(base, BR)], ins[0], sems[0]),
        1: pltpu.async_copy(emb_hbm.at[pl.ds(base + BR, BR)], ins[1], sems[1]),
    }
    ocps = {}
    for b in range(NBLK):
        if b + 2 < NBLK:
            cps[(b + 2) % 3] = pltpu.async_copy(
                emb_hbm.at[pl.ds(base + (b + 2) * BR, BR)], ins[(b + 2) % 3],
                sems[(b + 2) % 3])
        cps[b % 3].wait()
        if b >= 2:
            ocps[b % 2].wait()
        src = ins[b % 3]
        pk = pks[b % 2]

        def prow(r, carry):
            for rr in range(PU):
                for u in range(4):
                    row = PU * r + rr
                    wa = lax.bitcast_convert_type(
                        src[row, pl.ds(16 * u, 16)], jnp.int32)
                    wb = lax.bitcast_convert_type(
                        src[row, pl.ds(64 + 16 * u, 16)], jnp.int32)
                    ta = lax.shift_right_logical(wa + 0x8000, 16)
                    tb = (wb + 0x8000) & jnp.int32(-65536)
                    pk[row, pl.ds(16 * u, 16)] = ta | tb
            return carry

        lax.fori_loop(0, BR // PU, prow, 0)
        ocps[b % 2] = pltpu.async_copy(
            pk, out_hbm.at[pl.ds(base + b * BR, BR)], osems[b % 2])
    ocps[(NBLK - 2) % 2].wait()
    ocps[(NBLK - 1) % 2].wait()


@functools.partial(
    pl.kernel,
    mesh=_mesh,
    out_type=jax.ShapeDtypeStruct((SEGS, D), jnp.float32),
    compiler_params=pltpu.CompilerParams(use_tc_tiling_on_sc=False),
    scratch_types=[
        pltpu.VMEM((G, CHUNKS, K), jnp.int32),
        pltpu.VMEM((G, CHUNKS, K), jnp.int32),
        pltpu.VMEM((K, D // 2), jnp.int32),
        pltpu.VMEM((K, D // 2), jnp.int32),
        pltpu.VMEM((K, D // 2), jnp.int32),
        pltpu.VMEM((G, D), jnp.float32),
        pltpu.VMEM((G, D), jnp.float32),
        pltpu.SemaphoreType.DMA,
        pltpu.SemaphoreType.DMA,
        pltpu.SemaphoreType.DMA,
        pltpu.SemaphoreType.DMA,
        pltpu.SemaphoreType.DMA,
        pltpu.SemaphoreType.DMA,
        pltpu.SemaphoreType.DMA,
    ],
)
def _sc_pool(idx_hbm, emb_hbm, out_hbm, idx_a, idx_b, rows_a, rows_b, rows_c,
             ost_a, ost_b, sia, sib, sga, sgb, sgc, soa, sob):
    wid = lax.axis_index("s") * NC + lax.axis_index("c")
    base = wid * SEG_PER_W
    idxs = (idx_a, idx_b)
    rows = (rows_a, rows_b, rows_c)
    osts = (ost_a, ost_b)
    isems = (sia, sib)
    gsems = (sga, sgb, sgc)
    osems = (soa, sob)
    NCK = G * CHUNKS
    RU = 4
    icps = {}
    ocps = {}

    def process_group(g, q, first):
        s0 = base + g * G
        icps[q].wait()
        gn = jnp.minimum(g + 1, NGRP - 1)
        icps[1 - q] = pltpu.async_copy(
            idx_hbm.at[pl.ds(base + gn * G, G)], idxs[1 - q], isems[1 - q])
        if not first:
            ocps[q].wait()
        iv = idxs[q]
        ost = osts[q]
        cps = {
            0: pltpu.async_copy(emb_hbm.at[iv.at[0, 0]], rows[0], gsems[0]),
            1: pltpu.async_copy(emb_hbm.at[iv.at[0, 1]], rows[1], gsems[1]),
        }
        acc = None
        for t in range(NCK):
            seg, j = divmod(t, CHUNKS)
            if t + 2 < NCK:
                seg2, j2 = divmod(t + 2, CHUNKS)
                cps[(t + 2) % 3] = pltpu.async_copy(
                    emb_hbm.at[iv.at[seg2, j2]], rows[(t + 2) % 3],
                    gsems[(t + 2) % 3])
            cps[t % 3].wait()
            buf = rows[t % 3]
            if j == 0:
                acc = tuple(jnp.zeros((16,), jnp.float32) for _ in range(LANES))

            def red(m, a):
                a = list(a)
                for mm in range(RU):
                    for u in range(4):
                        w = buf[RU * m + mm, pl.ds(16 * u, 16)]
                        a[u] = a[u] + lax.bitcast_convert_type(
                            w << 16, jnp.float32)
                        a[4 + u] = a[4 + u] + lax.bitcast_convert_type(
                            w, jnp.float32)
                return tuple(a)

            acc = lax.fori_loop(0, K // RU, red, acc)
            if j == CHUNKS - 1:
                for u in range(4):
                    ost[seg, pl.ds(16 * u, 16)] = acc[u] * (1.0 / L)
                    ost[seg, pl.ds(64 + 16 * u, 16)] = acc[4 + u] * (1.0 / L)
        ocps[q] = pltpu.async_copy(ost, out_hbm.at[pl.ds(s0, G)], osems[q])

    icps[0] = pltpu.async_copy(idx_hbm.at[pl.ds(base, G)], idxs[0], isems[0])
    process_group(0, 0, True)
    process_group(1, 1, True)

    def body(p, carry):
        process_group(2 * p, 0, False)
        process_group(2 * p + 1, 1, False)
        return carry

    lax.fori_loop(1, NGRP // 2, body, 0)
    pltpu.make_async_copy(
        idx_hbm.at[pl.ds(base, G)], idxs[0], isems[0]).wait()
    for q in (0, 1):
        pltpu.make_async_copy(
            osts[q], out_hbm.at[pl.ds(base, G)], osems[q]).wait()


BB = 512
H = 256
OPAD = 128


def _mlp_body(p_ref, w1_ref, b1_ref, w2_ref, b2_ref, o_ref):
    p = p_ref[...]
    w1 = w1_ref[...]
    h = jnp.dot(p[0], w1[0:D], preferred_element_type=jnp.float32)
    h = h + jnp.dot(p[1], w1[D:2 * D], preferred_element_type=jnp.float32)
    h = h + jnp.dot(p[2], w1[2 * D:3 * D], preferred_element_type=jnp.float32)
    h = jnp.maximum(h + b1_ref[...], 0.0)
    o_ref[...] = jnp.dot(h, w2_ref[...],
                         preferred_element_type=jnp.float32) + b2_ref[...]


_mlp = pl.pallas_call(
    _mlp_body,
    grid=(B // BB,),
    in_specs=[
        pl.BlockSpec((NCH, BB, D), lambda i: (0, i, 0)),
        pl.BlockSpec((NCH * D, H), lambda i: (0, 0)),
        pl.BlockSpec((1, H), lambda i: (0, 0)),
        pl.BlockSpec((H, OPAD), lambda i: (0, 0)),
        pl.BlockSpec((1, OPAD), lambda i: (0, 0)),
    ],
    out_specs=pl.BlockSpec((BB, OPAD), lambda i: (i, 0)),
    out_shape=jax.ShapeDtypeStruct((B, OPAD), jnp.float32),
)


def kernel(x, emb, fc1_w, fc1_b, fc2_w, fc2_b):
    x = x.astype(jnp.int32)
    idx = jnp.concatenate([x[0], x[2], x[3]], axis=0).reshape(SEGS, CHUNKS, K)
    pooled = _sc_pool(idx, _sc_pack(emb))
    pooled3 = pooled.reshape(NCH, B, D)
    w1t = fc1_w.T
    b1 = fc1_b.reshape(1, H)
    w2t = jnp.zeros((H, OPAD), jnp.float32).at[:, :10].set(fc2_w.T)
    b2 = jnp.zeros((1, OPAD), jnp.float32).at[0, :10].set(fc2_b)
    out = _mlp(pooled3, w1t, b1, w2t, b2)
    return out[:, :10]
